# baseline (speedup 1.0000x reference)
import jax
import jax.numpy as jnp
from jax import lax
from jax.experimental import pallas as pl
from jax.experimental.pallas import tpu as pltpu
from jax.experimental.pallas import tpu_sc as plsc

VOCAB = 100000
BATCH = 4096
SEQ = 200

NC = 2
NS = 16
L = 16
NW = NC * NS
ROWS_PER_W = BATCH // NW
SUB = 16
NSUB = ROWS_PER_W // SUB
BLK = SUB * SEQ

SENT = VOCAB
SENT_VAL = -1e30
TPAD = VOCAB + L


def _sc_kernel(idx_hbm, table_hbm, out_hbm, table_sh, table_v,
               idx_a, idx_b, out_a, out_b, sia, sib, soa, sob):
    sid = lax.axis_index("s")
    wid = sid * NC + lax.axis_index("c")
    base = wid * NSUB * BLK
    idx_bufs = [(idx_a, sia), (idx_b, sib)]
    out_bufs = [(out_a, soa), (out_b, sob)]
    idx_cps = [pltpu.async_copy(idx_hbm.at[pl.ds(base, BLK)], idx_a, sia)]

    @pl.when(sid == 0)
    def _():
        pltpu.sync_copy(table_hbm, table_sh)

    plsc.subcore_barrier()
    pltpu.sync_copy(table_sh, table_v.at[pl.ds(0, VOCAB)])
    table_v[pl.ds(VOCAB, L)] = jnp.full((L,), SENT_VAL, jnp.float32)
    lane_off = lax.iota(jnp.int32, L) * SEQ
    out_pending = [None, None]

    for sb in range(NSUB):
        off = base + sb * BLK
        idx_v, _ = idx_bufs[sb % 2]
        out_v, osem = out_bufs[sb % 2]
        idx_cps[sb].wait()
        if sb + 1 < NSUB:
            nref, nsem = idx_bufs[(sb + 1) % 2]
            idx_cps.append(pltpu.async_copy(
                idx_hbm.at[pl.ds(off + BLK, BLK)], nref, nsem))
        if out_pending[sb % 2] is not None:
            out_pending[sb % 2].wait()
        for g in range(SUB // L):
            base_vec = lane_off + (g * L * SEQ)

            @plsc.parallel_loop(
                0, SEQ, unroll=16, carry=jnp.zeros((L,), jnp.float32))
            def ssum(j, acc):
                pos = base_vec + j
                iv = plsc.load_gather(idx_v, [pos])
                gv = plsc.load_gather(table_v, [iv])
                e = jnp.exp(gv)
                plsc.store_scatter(out_v, [pos], e)
                return acc + e

            inv = 1.0 / ssum

            @plsc.parallel_loop(0, SEQ, unroll=16)
            def _rescale(j):
                pos = base_vec + j
                e = plsc.load_gather(out_v, [pos])
                plsc.store_scatter(out_v, [pos], e * inv)

        out_pending[sb % 2] = pltpu.async_copy(
            out_v, out_hbm.at[pl.ds(off, BLK)], osem)

    for cp in out_pending:
        if cp is not None:
            cp.wait()


@jax.jit
def _log_odds_attention(idx_flat, logodds):
    mesh = plsc.VectorSubcoreMesh(core_axis_name="c", subcore_axis_name="s")
    return pl.kernel(
        _sc_kernel,
        mesh=mesh,
        compiler_params=pltpu.CompilerParams(needs_layout_passes=False),
        out_type=jax.ShapeDtypeStruct((BATCH * SEQ,), jnp.float32),
        scratch_types=[
            pltpu.VMEM_SHARED((VOCAB,), jnp.float32),
            pltpu.VMEM((TPAD,), jnp.float32),
            pltpu.VMEM((BLK,), jnp.int32),
            pltpu.VMEM((BLK,), jnp.int32),
            pltpu.VMEM((BLK,), jnp.float32),
            pltpu.VMEM((BLK,), jnp.float32),
            pltpu.SemaphoreType.DMA,
            pltpu.SemaphoreType.DMA,
            pltpu.SemaphoreType.DMA,
            pltpu.SemaphoreType.DMA,
        ],
    )(idx_flat, logodds)


def kernel(input_seq, hidden, masks, logodds):
    del hidden
    idx_flat = jnp.where(
        masks.reshape(-1), SENT, input_seq.reshape(-1).astype(jnp.int32))
    out_flat = _log_odds_attention(idx_flat, logodds)
    return out_flat.reshape(BATCH, SEQ)


# speedup vs baseline: 1.0289x; 1.0289x over previous
import jax
import jax.numpy as jnp
from jax import lax
from jax.experimental import pallas as pl
from jax.experimental.pallas import tpu as pltpu
from jax.experimental.pallas import tpu_sc as plsc

VOCAB = 100000
BATCH = 4096
SEQ = 200

NC = 2
NS = 16
L = 16
NW = NC * NS
ROWS_PER_W = BATCH // NW
SUB = 16
NSUB = ROWS_PER_W // SUB
BLK = SUB * SEQ

SENT = VOCAB
SENT_VAL = -1e30
TPAD = VOCAB + L


def _sc_kernel(idx_hbm, table_hbm, out_hbm, table_sh, table_v,
               idx_a, idx_b, out_a, out_b, sia, sib, soa, sob):
    sid = lax.axis_index("s")
    wid = sid * NC + lax.axis_index("c")
    base = wid * NSUB * BLK
    idx_bufs = [(idx_a, sia), (idx_b, sib)]
    out_bufs = [(out_a, soa), (out_b, sob)]
    idx_cps = [pltpu.async_copy(idx_hbm.at[pl.ds(base, BLK)], idx_a, sia)]

    @pl.when(sid == 0)
    def _():
        pltpu.sync_copy(table_hbm, table_sh)

    plsc.subcore_barrier()
    pltpu.sync_copy(table_sh, table_v.at[pl.ds(0, VOCAB)])
    table_v[pl.ds(VOCAB, L)] = jnp.full((L,), SENT_VAL, jnp.float32)
    lane_off = lax.iota(jnp.int32, L) * SEQ
    out_pending = [None, None]

    for sb in range(NSUB):
        off = base + sb * BLK
        idx_v, _ = idx_bufs[sb % 2]
        out_v, osem = out_bufs[sb % 2]
        idx_cps[sb].wait()
        if sb + 1 < NSUB:
            nref, nsem = idx_bufs[(sb + 1) % 2]
            idx_cps.append(pltpu.async_copy(
                idx_hbm.at[pl.ds(off + BLK, BLK)], nref, nsem))
        if out_pending[sb % 2] is not None:
            out_pending[sb % 2].wait()
        for g in range(SUB // L):
            base_vec = lane_off + (g * L * SEQ)

            @plsc.parallel_loop(
                0, SEQ, unroll=8, carry=jnp.zeros((L,), jnp.float32))
            def ssum(j, acc):
                pos = base_vec + j
                iv = plsc.load_gather(idx_v, [pos])
                gv = plsc.load_gather(table_v, [iv])
                e = jnp.exp(gv)
                plsc.store_scatter(out_v, [pos], e)
                return acc + e

            inv = 1.0 / ssum

            @plsc.parallel_loop(0, SEQ, unroll=8)
            def _rescale(j):
                pos = base_vec + j
                e = plsc.load_gather(out_v, [pos])
                plsc.store_scatter(out_v, [pos], e * inv)

        out_pending[sb % 2] = pltpu.async_copy(
            out_v, out_hbm.at[pl.ds(off, BLK)], osem)

    for cp in out_pending:
        if cp is not None:
            cp.wait()


@jax.jit
def _log_odds_attention(idx_flat, logodds):
    mesh = plsc.VectorSubcoreMesh(core_axis_name="c", subcore_axis_name="s")
    return pl.kernel(
        _sc_kernel,
        mesh=mesh,
        compiler_params=pltpu.CompilerParams(needs_layout_passes=False),
        out_type=jax.ShapeDtypeStruct((BATCH * SEQ,), jnp.float32),
        scratch_types=[
            pltpu.VMEM_SHARED((VOCAB,), jnp.float32),
            pltpu.VMEM((TPAD,), jnp.float32),
            pltpu.VMEM((BLK,), jnp.int32),
            pltpu.VMEM((BLK,), jnp.int32),
            pltpu.VMEM((BLK,), jnp.float32),
            pltpu.VMEM((BLK,), jnp.float32),
            pltpu.SemaphoreType.DMA,
            pltpu.SemaphoreType.DMA,
            pltpu.SemaphoreType.DMA,
            pltpu.SemaphoreType.DMA,
        ],
    )(idx_flat, logodds)


def kernel(input_seq, hidden, masks, logodds):
    del hidden
    idx_flat = jnp.where(
        masks.reshape(-1), SENT, input_seq.reshape(-1).astype(jnp.int32))
    out_flat = _log_odds_attention(idx_flat, logodds)
    return out_flat.reshape(BATCH, SEQ)
